# baseline (speedup 1.0000x reference)
import functools

import jax
import jax.numpy as jnp
from jax import lax
from jax.experimental import pallas as pl
from jax.experimental.pallas import tpu as pltpu
from jax.experimental.pallas import tpu_sc as plsc

K = 10
N = 8192
C = 256
TN = 512
TM = 512
SEG = 128
NSEG = N // SEG
SPB = TM // SEG

NC, NS = 2, 16
NW = NC * NS
IDXW3 = (N * K) // NW
CH3 = 512
IDXW5 = (N * K * 2) // NW
CH5 = 256


def _p1_kernel(x_ref, y_ref, sx_ref, sy_ref, s3_ref, g_ref):
    x = x_ref[...]
    y = y_ref[...]
    ab = lax.dot_general(x, y, (((1,), (1,)), ((), ())),
                         preferred_element_type=jnp.float32)
    sx = sx_ref[...]
    sy = sy_ref[...]
    sab = lax.dot_general(sx, sy, (((1,), (1,)), ((), ())),
                          preferred_element_type=jnp.float32)
    a2 = jnp.sum(x * x, axis=1, keepdims=True)
    b2 = jnp.sum(y * y, axis=1)[None, :]
    sa2 = jnp.sum(sx * sx, axis=1, keepdims=True)
    sb2 = jnp.sum(sy * sy, axis=1)[None, :]
    s = -((a2 + b2 - 2.0 * ab) + (sa2 + sb2 - 2.0 * sab))
    for t in range(SPB):
        blk = s[:, t * SEG:(t + 1) * SEG]
        s3_ref[t] = blk
        g_ref[0, :, t:t + 1] = jnp.max(blk, axis=1, keepdims=True)


def _p2_kernel(g_ref, flat_ref):
    g = jnp.concatenate([g_ref[j] for j in range(N // TM)],
                        axis=1)
    iota = lax.broadcasted_iota(jnp.int32, (TN, NSEG), 1)
    rows = (pl.program_id(0) * TN
            + lax.broadcasted_iota(jnp.int32, (TN, 1), 0))
    for k in range(K):
        m = jnp.max(g, axis=1, keepdims=True)
        p = jnp.min(jnp.where(g == m, iota, NSEG), axis=1, keepdims=True)
        flat_ref[:, k:k + 1] = p * N + rows
        g = jnp.where(iota == p, -jnp.inf, g)


def _p3_body(tab_ref, idx_ref, out_ref, idx_v, buf, sem):
    wid = lax.axis_index("s") * NC + lax.axis_index("c")
    base = wid * IDXW3
    pltpu.sync_copy(idx_ref.at[pl.ds(base, IDXW3)], idx_v)
    for ci in range(IDXW3 // CH3):
        pltpu.async_copy(tab_ref.at[idx_v.at[pl.ds(ci * CH3, CH3)]], buf,
                         sem).wait()
        pltpu.sync_copy(buf, out_ref.at[pl.ds(base + ci * CH3, CH3)])


def _p3(tab, idx):
    return pl.kernel(
        _p3_body,
        out_type=jax.ShapeDtypeStruct((N * K, SEG), jnp.float32),
        mesh=plsc.VectorSubcoreMesh(core_axis_name="c", subcore_axis_name="s"),
        scratch_types=[
            pltpu.VMEM((IDXW3,), jnp.int32),
            pltpu.VMEM((CH3, SEG), jnp.float32),
            pltpu.SemaphoreType.DMA,
        ],
    )(tab, idx)


def _p4_kernel(c_ref, flat_ref, score_ref, idx_ref, idx2_ref):
    flat = flat_ref[...]
    segs = flat // N
    iota_k = lax.broadcasted_iota(jnp.int32, (TN, K), 1)
    iota_s = lax.broadcasted_iota(jnp.int32, (TN, SEG), 1)
    cs = [c_ref[:, t, :] for t in range(K)]
    for k in range(K):
        ms, offs = [], []
        for t in range(K):
            mt = jnp.max(cs[t], axis=1, keepdims=True)
            ot = jnp.min(jnp.where(cs[t] == mt, iota_s, SEG), axis=1,
                         keepdims=True)
            ms.append(mt)
            offs.append(ot)
        m_slot = jnp.concatenate(ms, axis=1)
        off_slot = jnp.concatenate(offs, axis=1)
        m = jnp.max(m_slot, axis=1, keepdims=True)
        tstar = jnp.min(jnp.where(m_slot == m, iota_k, K), axis=1,
                        keepdims=True)
        sel = iota_k == tstar
        offstar = jnp.sum(jnp.where(sel, off_slot, 0), axis=1, keepdims=True)
        segstar = jnp.sum(jnp.where(sel, segs, 0), axis=1, keepdims=True)
        gid = segstar * SEG + offstar
        score_ref[:, k:k + 1] = m
        idx_ref[:, k:k + 1] = gid
        idx2_ref[:, 2 * k:2 * k + 1] = gid * 2
        idx2_ref[:, 2 * k + 1:2 * k + 2] = gid * 2 + 1
        for t in range(K):
            cs[t] = jnp.where((tstar == t) & (iota_s == offstar),
                              -jnp.inf, cs[t])


def _p5_body(y_ref, idx_ref, out_ref, idx_v, buf, sem):
    wid = lax.axis_index("s") * NC + lax.axis_index("c")
    base = wid * IDXW5
    pltpu.sync_copy(idx_ref.at[pl.ds(base, IDXW5)], idx_v)
    for ci in range(IDXW5 // CH5):
        pltpu.async_copy(y_ref.at[idx_v.at[pl.ds(ci * CH5, CH5)]], buf,
                         sem).wait()
        pltpu.sync_copy(buf, out_ref.at[pl.ds(base + ci * CH5, CH5)])


def _p5(y_lin, idx2):
    return pl.kernel(
        _p5_body,
        out_type=jax.ShapeDtypeStruct((N * K * 2, 128), jnp.float32),
        mesh=plsc.VectorSubcoreMesh(core_axis_name="c", subcore_axis_name="s"),
        scratch_types=[
            pltpu.VMEM((IDXW5,), jnp.int32),
            pltpu.VMEM((CH5, 128), jnp.float32),
            pltpu.SemaphoreType.DMA,
        ],
    )(y_lin, idx2)


TR = 64


def _diff_kernel(g_ref, x_ref, o_ref):
    xs = x_ref[...]
    for t in range(K):
        o_ref[:, t, :128] = g_ref[:, t, 0, :] - xs[:, :128]
        o_ref[:, t, 128:] = g_ref[:, t, 1, :] - xs[:, 128:]


def kernel(x, y, spatial):
    x2 = x[0]
    y2 = y[0]
    sp = spatial[0]
    spad = jnp.pad(sp, ((0, 0), (0, 6)))

    s3, g = pl.pallas_call(
        _p1_kernel,
        grid=(N // TN, N // TM),
        in_specs=[
            pl.BlockSpec((TN, C), lambda i, j: (i, 0)),
            pl.BlockSpec((TM, C), lambda i, j: (j, 0)),
            pl.BlockSpec((TN, 8), lambda i, j: (i, 0)),
            pl.BlockSpec((TM, 8), lambda i, j: (j, 0)),
        ],
        out_specs=[
            pl.BlockSpec((SPB, TN, SEG), lambda i, j: (j, i, 0)),
            pl.BlockSpec((1, TN, SPB), lambda i, j: (j, i, 0)),
        ],
        out_shape=[
            jax.ShapeDtypeStruct((NSEG, N, SEG), jnp.float32),
            jax.ShapeDtypeStruct((N // TM, N, SPB), jnp.float32),
        ],
    )(x2, y2, spad, spad)

    flat = pl.pallas_call(
        _p2_kernel,
        grid=(N // TN,),
        in_specs=[pl.BlockSpec((N // TM, TN, SPB), lambda i: (0, i, 0))],
        out_specs=pl.BlockSpec((TN, K), lambda i: (i, 0)),
        out_shape=jax.ShapeDtypeStruct((N, K), jnp.int32),
    )(g)

    cand = _p3(s3.reshape(NSEG * N, SEG), flat.reshape(N * K))

    score_k, idx_k, idx2 = pl.pallas_call(
        _p4_kernel,
        grid=(N // TN,),
        in_specs=[
            pl.BlockSpec((TN, K, SEG), lambda i: (i, 0, 0)),
            pl.BlockSpec((TN, K), lambda i: (i, 0)),
        ],
        out_specs=[
            pl.BlockSpec((TN, K), lambda i: (i, 0)),
            pl.BlockSpec((TN, K), lambda i: (i, 0)),
            pl.BlockSpec((TN, 2 * K), lambda i: (i, 0)),
        ],
        out_shape=[
            jax.ShapeDtypeStruct((N, K), jnp.float32),
            jax.ShapeDtypeStruct((N, K), jnp.int32),
            jax.ShapeDtypeStruct((N, 2 * K), jnp.int32),
        ],
    )(cand.reshape(N, K, SEG), flat)

    y_lin = y2.reshape(N * 2, 128)
    gathered = _p5(y_lin, idx2.reshape(N * K * 2))
    g4 = gathered.reshape(N, K, 2, 128)

    diff_patch = pl.pallas_call(
        _diff_kernel,
        grid=(N // TR,),
        in_specs=[
            pl.BlockSpec((TR, K, 2, 128), lambda i: (i, 0, 0, 0)),
            pl.BlockSpec((TR, C), lambda i: (i, 0)),
        ],
        out_specs=pl.BlockSpec((TR, K, C), lambda i: (i, 0, 0)),
        out_shape=jax.ShapeDtypeStruct((N, K, C), jnp.float32),
    )(g4, x2)

    return score_k[None], idx_k[None], diff_patch[None]


# speedup vs baseline: 1.7676x; 1.7676x over previous
import functools

import jax
import jax.numpy as jnp
from jax import lax
from jax.experimental import pallas as pl
from jax.experimental.pallas import tpu as pltpu
from jax.experimental.pallas import tpu_sc as plsc

K = 10
N = 8192
C = 256
TN = 512
TN1 = 1024
TM = 512
SEG = 128
NSEG = N // SEG
SPB = TM // SEG

NC, NS = 2, 16
NW = NC * NS
IDXW3 = (N * K) // NW
CH3 = 512
IDXW5 = (N * K * 2) // NW
CH5 = 256


def _p1_kernel(x_ref, y_ref, sx_ref, sy_ref, s3_ref, g_ref):
    x = x_ref[...]
    y = y_ref[...]
    ab = lax.dot_general(x, y, (((1,), (1,)), ((), ())),
                         preferred_element_type=jnp.float32)
    sx = sx_ref[...]
    sy = sy_ref[...]
    sab = lax.dot_general(sx, sy, (((1,), (1,)), ((), ())),
                          preferred_element_type=jnp.float32)
    a2 = jnp.sum(x * x, axis=1, keepdims=True)
    b2 = jnp.sum(y * y, axis=1)[None, :]
    sa2 = jnp.sum(sx * sx, axis=1, keepdims=True)
    sb2 = jnp.sum(sy * sy, axis=1)[None, :]
    s = -((a2 + b2 - 2.0 * ab) + (sa2 + sb2 - 2.0 * sab))
    for t in range(SPB):
        blk = s[:, t * SEG:(t + 1) * SEG]
        s3_ref[t] = blk
        g_ref[0, :, t:t + 1] = jnp.max(blk, axis=1, keepdims=True)


def _p2_kernel(g_ref, flat_ref):
    g = jnp.concatenate([g_ref[j] for j in range(N // TM)],
                        axis=1)
    iota = lax.broadcasted_iota(jnp.int32, (TN, NSEG), 1)
    rows = (pl.program_id(0) * TN
            + lax.broadcasted_iota(jnp.int32, (TN, 1), 0))
    for k in range(K):
        m = jnp.max(g, axis=1, keepdims=True)
        p = jnp.min(jnp.where(g == m, iota, NSEG), axis=1, keepdims=True)
        flat_ref[:, k:k + 1] = p * N + rows
        g = jnp.where(iota == p, -jnp.inf, g)


def _p3_body(tab_ref, idx_ref, out_ref, idx_v, buf, sem):
    wid = lax.axis_index("s") * NC + lax.axis_index("c")
    base = wid * IDXW3
    pltpu.sync_copy(idx_ref.at[pl.ds(base, IDXW3)], idx_v)
    for ci in range(IDXW3 // CH3):
        pltpu.async_copy(tab_ref.at[idx_v.at[pl.ds(ci * CH3, CH3)]], buf,
                         sem).wait()
        pltpu.sync_copy(buf, out_ref.at[pl.ds(base + ci * CH3, CH3)])


def _p3(tab, idx):
    return pl.kernel(
        _p3_body,
        out_type=jax.ShapeDtypeStruct((N * K, SEG), jnp.float32),
        mesh=plsc.VectorSubcoreMesh(core_axis_name="c", subcore_axis_name="s"),
        scratch_types=[
            pltpu.VMEM((IDXW3,), jnp.int32),
            pltpu.VMEM((CH3, SEG), jnp.float32),
            pltpu.SemaphoreType.DMA,
        ],
    )(tab, idx)


def _p4_kernel(c_ref, flat_ref, score_ref, idx_ref, idx2_ref):
    flat = flat_ref[...]
    segs = flat // N
    iota_k = lax.broadcasted_iota(jnp.int32, (TN, K), 1)
    W = K * SEG
    cw = jnp.concatenate([c_ref[:, t, :] for t in range(K)], axis=1)
    iota_w = lax.broadcasted_iota(jnp.int32, (TN, W), 1)
    for k in range(K):
        m = jnp.max(cw, axis=1, keepdims=True)
        pos = jnp.min(jnp.where(cw == m, iota_w, W), axis=1, keepdims=True)
        tstar = pos // SEG
        offstar = pos % SEG
        segstar = jnp.sum(jnp.where(iota_k == tstar, segs, 0), axis=1,
                          keepdims=True)
        gid = segstar * SEG + offstar
        score_ref[:, k:k + 1] = m
        idx_ref[:, k:k + 1] = gid
        idx2_ref[:, 2 * k:2 * k + 1] = gid * 2
        idx2_ref[:, 2 * k + 1:2 * k + 2] = gid * 2 + 1
        cw = jnp.where(iota_w == pos, -jnp.inf, cw)


def _p5_body(y_ref, idx_ref, out_ref, idx_v, buf, sem):
    wid = lax.axis_index("s") * NC + lax.axis_index("c")
    base = wid * IDXW5
    pltpu.sync_copy(idx_ref.at[pl.ds(base, IDXW5)], idx_v)
    for ci in range(IDXW5 // CH5):
        pltpu.async_copy(y_ref.at[idx_v.at[pl.ds(ci * CH5, CH5)]], buf,
                         sem).wait()
        pltpu.sync_copy(buf, out_ref.at[pl.ds(base + ci * CH5, CH5)])


def _p5(y_lin, idx2):
    return pl.kernel(
        _p5_body,
        out_type=jax.ShapeDtypeStruct((N * K * 2, 128), jnp.float32),
        mesh=plsc.VectorSubcoreMesh(core_axis_name="c", subcore_axis_name="s"),
        scratch_types=[
            pltpu.VMEM((IDXW5,), jnp.int32),
            pltpu.VMEM((CH5, 128), jnp.float32),
            pltpu.SemaphoreType.DMA,
        ],
    )(y_lin, idx2)


TR = 64


def _diff_kernel(g_ref, x_ref, o_ref):
    xs = x_ref[...]
    for t in range(K):
        o_ref[:, t, :128] = g_ref[:, t, 0, :] - xs[:, :128]
        o_ref[:, t, 128:] = g_ref[:, t, 1, :] - xs[:, 128:]


def kernel(x, y, spatial):
    x2 = x[0]
    y2 = y[0]
    sp = spatial[0]
    spad = jnp.pad(sp, ((0, 0), (0, 6)))

    s3, g = pl.pallas_call(
        _p1_kernel,
        grid=(N // TN1, N // TM),
        in_specs=[
            pl.BlockSpec((TN1, C), lambda i, j: (i, 0)),
            pl.BlockSpec((TM, C), lambda i, j: (j, 0)),
            pl.BlockSpec((TN1, 8), lambda i, j: (i, 0)),
            pl.BlockSpec((TM, 8), lambda i, j: (j, 0)),
        ],
        out_specs=[
            pl.BlockSpec((SPB, TN1, SEG), lambda i, j: (j, i, 0)),
            pl.BlockSpec((1, TN1, SPB), lambda i, j: (j, i, 0)),
        ],
        out_shape=[
            jax.ShapeDtypeStruct((NSEG, N, SEG), jnp.float32),
            jax.ShapeDtypeStruct((N // TM, N, SPB), jnp.float32),
        ],
    )(x2, y2, spad, spad)

    flat = pl.pallas_call(
        _p2_kernel,
        grid=(N // TN,),
        in_specs=[pl.BlockSpec((N // TM, TN, SPB), lambda i: (0, i, 0))],
        out_specs=pl.BlockSpec((TN, K), lambda i: (i, 0)),
        out_shape=jax.ShapeDtypeStruct((N, K), jnp.int32),
    )(g)

    cand = _p3(s3.reshape(NSEG * N, SEG), flat.reshape(N * K))

    score_k, idx_k, idx2 = pl.pallas_call(
        _p4_kernel,
        grid=(N // TN,),
        in_specs=[
            pl.BlockSpec((TN, K, SEG), lambda i: (i, 0, 0)),
            pl.BlockSpec((TN, K), lambda i: (i, 0)),
        ],
        out_specs=[
            pl.BlockSpec((TN, K), lambda i: (i, 0)),
            pl.BlockSpec((TN, K), lambda i: (i, 0)),
            pl.BlockSpec((TN, 2 * K), lambda i: (i, 0)),
        ],
        out_shape=[
            jax.ShapeDtypeStruct((N, K), jnp.float32),
            jax.ShapeDtypeStruct((N, K), jnp.int32),
            jax.ShapeDtypeStruct((N, 2 * K), jnp.int32),
        ],
    )(cand.reshape(N, K, SEG), flat)

    y_lin = y2.reshape(N * 2, 128)
    gathered = _p5(y_lin, idx2.reshape(N * K * 2))
    g4 = gathered.reshape(N, K, 2, 128)

    diff_patch = pl.pallas_call(
        _diff_kernel,
        grid=(N // TR,),
        in_specs=[
            pl.BlockSpec((TR, K, 2, 128), lambda i: (i, 0, 0, 0)),
            pl.BlockSpec((TR, C), lambda i: (i, 0)),
        ],
        out_specs=pl.BlockSpec((TR, K, C), lambda i: (i, 0, 0)),
        out_shape=jax.ShapeDtypeStruct((N, K, C), jnp.float32),
    )(g4, x2)

    return score_k[None], idx_k[None], diff_patch[None]
